# Initial kernel scaffold; baseline (speedup 1.0000x reference)
import jax
import jax.numpy as jnp
from jax.experimental import pallas as pl


def kernel(x, branch0_wk, branch0_b, branch1_0_wk, branch1_0_b, branch1_1_wk, branch1_1_b, branch1_2_wk, branch1_2_b):
    raise NotImplementedError("write your pallas kernel here")


# speedup vs baseline: 8.5055x; 8.5055x over previous
import jax
import jax.numpy as jnp
from jax.experimental import pallas as pl
from jax.experimental.pallas import tpu as pltpu

_H = 27
_HO = 13
_P = 14
_MP = _P * _P
_M2 = _HO * _HO

_TAP = {0: (0, 0), 1: (1, 0), 2: (0, 1)}


def _mega_kernel(x_ref, w0_ref, b0_ref, w1_ref, b1_ref, w2_ref, b2_ref,
                 w3_ref, b3_ref, o_ref):
    xflat = x_ref[0].reshape(4 * _MP, 384)

    y1 = jnp.dot(xflat.astype(jnp.bfloat16), w1_ref[...],
                 preferred_element_type=jnp.float32)
    y1 = jnp.maximum(y1 + b1_ref[...], 0.0)
    y1 = y1.reshape(4, _P, _P, 192)

    pad = []
    for u in range(2):
        rows = _P if u == 0 else _P - 1
        prow = []
        for v in range(2):
            cols = _P if v == 0 else _P - 1
            t = y1[2 * u + v][:rows, :cols, :].astype(jnp.bfloat16)
            zt = jnp.zeros((1, cols, 192), jnp.bfloat16)
            zb = jnp.zeros((16 - rows - 1, cols, 192), jnp.bfloat16)
            t = jnp.concatenate([zt, t, zb], axis=0)
            zl = jnp.zeros((16, 1, 192), jnp.bfloat16)
            zr = jnp.zeros((16, 16 - cols - 1, 192), jnp.bfloat16)
            prow.append(jnp.concatenate([zl, t, zr], axis=1))
        pad.append(prow)

    acc = jnp.zeros((4 * _MP, 224), jnp.float32)
    for dh in range(3):
        for dw in range(3):
            parts = []
            for u in range(2):
                e = u + dh - 1
                up, po = e % 2, (e - (e % 2)) // 2 + 1
                for v in range(2):
                    f = v + dw - 1
                    vp, qo = f % 2, (f - (f % 2)) // 2 + 1
                    t = pad[up][vp][po:po + _P, qo:qo + _P, :]
                    parts.append(t.reshape(_MP, 192))
            acc = acc + jnp.dot(jnp.concatenate(parts, axis=0),
                                w2_ref[dh * 3 + dw],
                                preferred_element_type=jnp.float32)
    y2 = jnp.maximum(acc + b2_ref[...], 0.0)
    y2 = y2.astype(jnp.bfloat16).reshape(4, _P, _P, 224)

    acc1 = jnp.zeros((_M2, 256), jnp.float32)
    for dh in range(3):
        up, po = _TAP[dh]
        for dw in range(3):
            vp, qo = _TAP[dw]
            t = y2[2 * up + vp][po:po + _HO, qo:qo + _HO, :]
            acc1 = acc1 + jnp.dot(t.reshape(_M2, 224),
                                  w3_ref[dh * 3 + dw],
                                  preferred_element_type=jnp.float32)
    x1 = jnp.maximum(acc1 + b3_ref[...], 0.0)

    xpl = x_ref[0]
    acc0 = jnp.zeros((_M2, 384), jnp.float32)
    mx = None
    for dh in range(3):
        up, po = _TAP[dh]
        for dw in range(3):
            vp, qo = _TAP[dw]
            tf = xpl[up, vp, po:po + _HO, qo:qo + _HO, :].reshape(_M2, 384)
            acc0 = acc0 + jnp.dot(tf.astype(jnp.bfloat16),
                                  w0_ref[dh * 3 + dw],
                                  preferred_element_type=jnp.float32)
            mx = tf if mx is None else jnp.maximum(mx, tf)
    x0 = jnp.maximum(acc0 + b0_ref[...], 0.0)

    o_ref[0, :, 0:384] = x0
    o_ref[0, :, 384:640] = x1
    o_ref[0, :, 640:1024] = mx


def kernel(x, branch0_wk, branch0_b, branch1_0_wk, branch1_0_b,
           branch1_1_wk, branch1_1_b, branch1_2_wk, branch1_2_b):
    N = x.shape[0]
    xp = jnp.pad(x.astype(jnp.float32), ((0, 0), (0, 0), (0, 1), (0, 1)))
    xp = xp.reshape(N, 384, _P, 2, _P, 2)
    xs2d = jnp.transpose(xp, (0, 3, 5, 2, 4, 1))

    out = pl.pallas_call(
        _mega_kernel,
        out_shape=jax.ShapeDtypeStruct((N, _M2, 1024), jnp.float32),
        grid_spec=pltpu.PrefetchScalarGridSpec(
            num_scalar_prefetch=0,
            grid=(N,),
            in_specs=[
                pl.BlockSpec((1, 2, 2, _P, _P, 384),
                             lambda n: (n, 0, 0, 0, 0, 0)),
                pl.BlockSpec((9, 384, 384), lambda n: (0, 0, 0)),
                pl.BlockSpec((1, 384), lambda n: (0, 0)),
                pl.BlockSpec((384, 192), lambda n: (0, 0)),
                pl.BlockSpec((1, 192), lambda n: (0, 0)),
                pl.BlockSpec((9, 192, 224), lambda n: (0, 0, 0)),
                pl.BlockSpec((1, 224), lambda n: (0, 0)),
                pl.BlockSpec((9, 224, 256), lambda n: (0, 0, 0)),
                pl.BlockSpec((1, 256), lambda n: (0, 0)),
            ],
            out_specs=pl.BlockSpec((1, _M2, 1024), lambda n: (n, 0, 0)),
        ),
        compiler_params=pltpu.CompilerParams(
            dimension_semantics=("parallel",)),
    )(xs2d, branch0_wk, branch0_b.reshape(1, 384),
      branch1_0_wk, branch1_0_b.reshape(1, 192),
      branch1_1_wk, branch1_1_b.reshape(1, 224),
      branch1_2_wk, branch1_2_b.reshape(1, 256))

    out = out.reshape(N, _HO, _HO, 1024)
    return jnp.transpose(out, (0, 3, 1, 2))


# speedup vs baseline: 12.2094x; 1.4355x over previous
import numpy as np

import jax
import jax.numpy as jnp
from jax.experimental import pallas as pl
from jax.experimental.pallas import tpu as pltpu

_HO = 13
_F = 256
_MO = 208
_PB = 32

_TAP = {0: (0, 0), 1: (1, 0), 2: (0, 1)}


def _mask_np():
    m = np.zeros((2, 2, _F, 192), np.float32)
    for u in range(2):
        for v in range(2):
            pm = 14 if u == 0 else 13
            qm = 14 if v == 0 else 13
            m2 = np.zeros((16, 16), np.float32)
            m2[:pm, :qm] = 1.0
            m[u, v] = np.broadcast_to(m2.reshape(_F, 1), (_F, 192))
    return m


_MASK = _mask_np()


def _mega_kernel(x_ref, mask_ref, w0_ref, b0_ref, w1_ref, b1_ref, w2_ref,
                 b2_ref, w3_ref, b3_ref, o_ref, p_ref, y2_ref):

    zhead = jnp.zeros((_PB, 192), jnp.bfloat16)
    for u in range(2):
        for v in range(2):
            y = jnp.dot(x_ref[0, u, v], w1_ref[...],
                        preferred_element_type=jnp.float32)
            y = jnp.maximum(y + b1_ref[...], 0.0)
            p_ref[u, v, _PB:_PB + _F, :] = (
                y.astype(jnp.bfloat16) * mask_ref[u, v])
            p_ref[u, v, 0:_PB, :] = zhead
            p_ref[u, v, _PB + _F:, :] = zhead

    for u in range(2):
        for v in range(2):
            acc = jnp.zeros((_F, 224), jnp.float32)
            for dh in range(3):
                e = u + dh - 1
                up, pa = e % 2, (e - (e % 2)) // 2
                for dw in range(3):
                    f = v + dw - 1
                    vp, qa = f % 2, (f - (f % 2)) // 2
                    ofs = _PB + 16 * pa + qa
                    acc = acc + jnp.dot(p_ref[up, vp, ofs:ofs + _F, :],
                                        w2_ref[dh * 3 + dw],
                                        preferred_element_type=jnp.float32)
            y2 = jnp.maximum(acc + b2_ref[...], 0.0)
            y2_ref[u, v] = y2.astype(jnp.bfloat16)

    acc1 = jnp.zeros((_MO, 256), jnp.float32)
    for dh in range(3):
        up, pa = _TAP[dh]
        for dw in range(3):
            vp, qa = _TAP[dw]
            ofs = 16 * pa + qa
            acc1 = acc1 + jnp.dot(y2_ref[up, vp, ofs:ofs + _MO, :],
                                  w3_ref[dh * 3 + dw],
                                  preferred_element_type=jnp.float32)
    x1 = jnp.maximum(acc1 + b3_ref[...], 0.0)

    acc0 = jnp.zeros((_MO, 384), jnp.float32)
    mx = None
    for dh in range(3):
        up, pa = _TAP[dh]
        for dw in range(3):
            vp, qa = _TAP[dw]
            ofs = 16 * pa + qa
            tb = x_ref[0, up, vp, ofs:ofs + _MO, :]
            acc0 = acc0 + jnp.dot(tb, w0_ref[dh * 3 + dw],
                                  preferred_element_type=jnp.float32)
            mx = tb if mx is None else jnp.maximum(mx, tb)
    x0 = jnp.maximum(acc0 + b0_ref[...], 0.0)

    o_ref[0, :, 0:384] = x0
    o_ref[0, :, 384:640] = x1
    o_ref[0, :, 640:1024] = mx.astype(jnp.float32)


def kernel(x, branch0_wk, branch0_b, branch1_0_wk, branch1_0_b,
           branch1_1_wk, branch1_1_b, branch1_2_wk, branch1_2_b):
    N = x.shape[0]
    xp = jnp.pad(x.astype(jnp.bfloat16), ((0, 0), (0, 0), (0, 5), (0, 5)))
    xp = xp.reshape(N, 384, 16, 2, 16, 2)
    xs2d = jnp.transpose(xp, (0, 3, 5, 2, 4, 1)).reshape(N, 2, 2, _F, 384)
    mask = jnp.asarray(_MASK, jnp.bfloat16)

    out = pl.pallas_call(
        _mega_kernel,
        out_shape=jax.ShapeDtypeStruct((N, _MO, 1024), jnp.float32),
        grid_spec=pltpu.PrefetchScalarGridSpec(
            num_scalar_prefetch=0,
            grid=(N,),
            in_specs=[
                pl.BlockSpec((1, 2, 2, _F, 384), lambda n: (n, 0, 0, 0, 0)),
                pl.BlockSpec((2, 2, _F, 192), lambda n: (0, 0, 0, 0)),
                pl.BlockSpec((9, 384, 384), lambda n: (0, 0, 0)),
                pl.BlockSpec((1, 384), lambda n: (0, 0)),
                pl.BlockSpec((384, 192), lambda n: (0, 0)),
                pl.BlockSpec((1, 192), lambda n: (0, 0)),
                pl.BlockSpec((9, 192, 224), lambda n: (0, 0, 0)),
                pl.BlockSpec((1, 224), lambda n: (0, 0)),
                pl.BlockSpec((9, 224, 256), lambda n: (0, 0, 0)),
                pl.BlockSpec((1, 256), lambda n: (0, 0)),
            ],
            out_specs=pl.BlockSpec((1, _MO, 1024), lambda n: (n, 0, 0)),
            scratch_shapes=[
                pltpu.VMEM((2, 2, 320, 192), jnp.bfloat16),
                pltpu.VMEM((2, 2, _F, 224), jnp.bfloat16),
            ],
        ),
        compiler_params=pltpu.CompilerParams(
            dimension_semantics=("parallel",)),
    )(xs2d, mask, branch0_wk, branch0_b.reshape(1, 384),
      branch1_0_wk, branch1_0_b.reshape(1, 192),
      branch1_1_wk, branch1_1_b.reshape(1, 224),
      branch1_2_wk, branch1_2_b.reshape(1, 256))

    out = out.reshape(N, _HO, 16, 1024)[:, :, :_HO, :]
    return jnp.transpose(out, (0, 3, 1, 2))
